# baseline (speedup 1.0000x reference)
import functools

import numpy as np
import jax
import jax.numpy as jnp
from jax import lax
from jax.experimental import pallas as pl
from jax.experimental.pallas import tpu as pltpu
from jax.experimental.pallas import tpu_sc as plsc

B, F, V, D = 16384, 26, 100000, 16
R = F * V

NC, NS, L = 2, 16, 16
NW = NC * NS

PIECE = 1920
NPIECE = R // PIECE
KFULL = NPIECE // NW
KREM = NPIECE - KFULL * NW
NIB = 2
TAILOFF = NPIECE * PIECE
TAIL = 256
LASTOFF = TAILOFF + TAIL
LASTN = R - LASTOFF

RPW = B // NW
CHUNK = 128
NCHUNK = RPW // CHUNK
CI = CHUNK * F
PER = 13
NPERIOD = CI // (PER * L)
GATHER_ROWS = 128
NGATH = CI // GATHER_ROWS

_OFFS = ((np.arange(PER * L, dtype=np.int64) % F) * V).astype(np.int32)

_mesh = plsc.VectorSubcoreMesh(
    core_axis_name="c", subcore_axis_name="s", num_cores=NC, num_subcores=NS
)


@functools.partial(
    pl.kernel,
    out_type=jax.ShapeDtypeStruct((R * D,), jnp.float32),
    mesh=_mesh,
    compiler_params=pltpu.CompilerParams(
        needs_layout_passes=False, use_tc_tiling_on_sc=True
    ),
    scratch_types=[
        pltpu.VMEM((D, PIECE), jnp.float32),
        pltpu.VMEM((D, PIECE), jnp.float32),
        pltpu.VMEM((PIECE * D,), jnp.float32),
        pltpu.VMEM((PIECE * D,), jnp.float32),
        pltpu.SemaphoreType.DMA,
        pltpu.SemaphoreType.DMA,
        pltpu.SemaphoreType.DMA,
        pltpu.SemaphoreType.DMA,
    ],
)
def _detile(tt_hbm, last_hbm, lin_hbm, inb0, inb1, outb0, outb1,
            sin0, sin1, sout0, sout1):
    wid = lax.axis_index("s") * NC + lax.axis_index("c")
    inbufs, outbufs = (inb0, inb1), (outb0, outb1)
    sins, souts = (sin0, sin1), (sout0, sout1)
    base16 = lax.iota(jnp.int32, L) * D
    colidx = [base16 + d for d in range(D)]

    def start_in(k):
        p = wid + k * NW
        return pltpu.async_copy(
            tt_hbm.at[:, pl.ds(p * PIECE, PIECE)], inbufs[k % NIB], sins[k % NIB]
        )

    def transpose(k, nrows=PIECE):
        ib, ob = inbufs[k % NIB], outbufs[k % 2]

        @pl.loop(0, nrows // L)
        def _blk(g):
            rbase = g * L
            dst = ob.at[pl.ds(rbase * D, L * D)]
            vs = [ib[d, pl.ds(rbase, L)] for d in range(D)]
            for d in range(D):
                plsc.store_scatter(dst, [colidx[d]], vs[d])

    def start_out(k):
        p = wid + k * NW
        return pltpu.async_copy(
            outbufs[k % 2], lin_hbm.at[pl.ds(p * PIECE * D, PIECE * D)],
            souts[k % 2],
        )

    h_in = {0: start_in(0)}
    h_out = {}
    for k in range(KFULL):
        if k + 1 < KFULL:
            h_in[k + 1] = start_in(k + 1)
        h_in.pop(k).wait()
        if k - 2 in h_out:
            h_out.pop(k - 2).wait()
        transpose(k)
        h_out[k] = start_out(k)

    h_out.pop(KFULL - 2).wait()
    h_out.pop(KFULL - 1).wait()

    @pl.when(wid < KREM)
    def _():
        p = NPIECE - KREM + wid
        pltpu.async_copy(
            tt_hbm.at[:, pl.ds(p * PIECE, PIECE)], inb0, sin0
        ).wait()
        transpose(0)
        pltpu.async_copy(
            outb0, lin_hbm.at[pl.ds(p * PIECE * D, PIECE * D)], sout0
        ).wait()

    @pl.when(wid == KREM)
    def _():
        pltpu.async_copy(
            tt_hbm.at[:, pl.ds(TAILOFF, TAIL)],
            inb0.at[:, pl.ds(0, TAIL)], sin0,
        ).wait()
        transpose(0, nrows=TAIL)
        pltpu.async_copy(
            outb0.at[pl.ds(0, TAIL * D)],
            lin_hbm.at[pl.ds(TAILOFF * D, TAIL * D)], sout0,
        ).wait()

    @pl.when(wid == KREM + 1)
    def _():
        pltpu.async_copy(
            last_hbm, outb0.at[pl.ds(0, LASTN * D)], sin0
        ).wait()
        pltpu.async_copy(
            outb0.at[pl.ds(0, LASTN * D)],
            lin_hbm.at[pl.ds(LASTOFF * D, LASTN * D)], sout0,
        ).wait()


@functools.partial(
    pl.kernel,
    out_type=jax.ShapeDtypeStruct((B,), jnp.float32),
    mesh=_mesh,
    compiler_params=pltpu.CompilerParams(
        needs_layout_passes=False, use_tc_tiling_on_sc=False
    ),
    scratch_types=[
        pltpu.VMEM((PER * L,), jnp.int32),
        pltpu.VMEM((CI,), jnp.int32),
        pltpu.VMEM((CI,), jnp.int32),
        pltpu.VMEM((CI,), jnp.int32),
        pltpu.VMEM((CI, D), jnp.float32),
        pltpu.VMEM((CI, D), jnp.float32),
        pltpu.VMEM((RPW,), jnp.float32),
        pltpu.SemaphoreType.DMA,
        pltpu.SemaphoreType.DMA,
    ],
)
def _fm_cross(table_hbm, x_hbm, offs_hbm, out_hbm,
              offs_v, xbuf, idxb0, idxb1, rows0, rows1, out_v, sem0, sem1):
    wid = lax.axis_index("s") * NC + lax.axis_index("c")
    tbase = wid * (RPW * F)
    pltpu.sync_copy(offs_hbm, offs_v)
    offs = [offs_v[pl.ds(p * L, L)] for p in range(PER)]
    idxbs, rowss, sems = (idxb0, idxb1), (rows0, rows1), (sem0, sem1)

    def build_and_fire(c):
        idxb, rows, sem = idxbs[c % 2], rowss[c % 2], sems[c % 2]
        base = tbase + c * CI
        pltpu.sync_copy(x_hbm.at[pl.ds(base, CI)], xbuf)

        @pl.loop(0, NPERIOD)
        def _b(it):
            xb = it * (PER * L)
            xs = [xbuf[pl.ds(xb + p * L, L)] for p in range(PER)]
            for p in range(PER):
                idxb[pl.ds(xb + p * L, L)] = xs[p] + offs[p]

        for j in range(NGATH):
            pltpu.async_copy(
                table_hbm.at[idxb.at[pl.ds(j * GATHER_ROWS, GATHER_ROWS)]],
                rows.at[pl.ds(j * GATHER_ROWS, GATHER_ROWS)],
                sem,
            )

    def drain(c):
        idxb, rows, sem = idxbs[c % 2], rowss[c % 2], sems[c % 2]
        for j in range(NGATH):
            pltpu.make_async_copy(
                table_hbm.at[idxb.at[pl.ds(j * GATHER_ROWS, GATHER_ROWS)]],
                rows.at[pl.ds(j * GATHER_ROWS, GATHER_ROWS)],
                sem,
            ).wait()

    def compute(c):
        rows = rowss[c % 2]

        @pl.loop(0, CHUNK // L)
        def _rowblk(b):
            lane = lax.iota(jnp.int32, L)
            res = jnp.zeros((L,), jnp.float32)
            for k in range(L):
                rb = (b * L + k) * F
                vs = [rows[rb + f] for f in range(F)]
                s = vs[0]
                ss = vs[0] * vs[0]
                for f in range(1, F):
                    s = s + vs[f]
                    ss = ss + vs[f] * vs[f]
                val = 0.5 * jnp.sum(s * s - ss)
                res = jnp.where(lane == k, val, res)
            out_v[pl.ds(c * CHUNK + b * L, L)] = res

    build_and_fire(0)
    for c in range(NCHUNK):
        if c + 1 < NCHUNK:
            build_and_fire(c + 1)
        drain(c)
        compute(c)

    pltpu.sync_copy(out_v, out_hbm.at[pl.ds(wid * RPW, RPW)])


def kernel(x, table):
    xflat = x.astype(jnp.int32).reshape(B * F)
    last = table[LASTOFF:].reshape(LASTN * D)
    lin = _detile(jnp.swapaxes(table, 0, 1), last)
    out = _fm_cross(lin.reshape(R, D), xflat, jnp.asarray(_OFFS))
    return out.reshape(B, 1)


# speedup vs baseline: 1.0146x; 1.0146x over previous
import functools

import numpy as np
import jax
import jax.numpy as jnp
from jax import lax
from jax.experimental import pallas as pl
from jax.experimental.pallas import tpu as pltpu
from jax.experimental.pallas import tpu_sc as plsc

B, F, V, D = 16384, 26, 100000, 16
R = F * V

NC, NS, L = 2, 16, 16
NW = NC * NS

PIECE = 1536
NPIECE = R // PIECE
KFULL = NPIECE // NW
KREM = NPIECE - KFULL * NW
NIB = 3
TAILOFF = NPIECE * PIECE
TAIL = 1024
LASTOFF = TAILOFF + TAIL
LASTN = R - LASTOFF

RPW = B // NW
CHUNK = 128
NCHUNK = RPW // CHUNK
CI = CHUNK * F
PER = 13
NPERIOD = CI // (PER * L)
GATHER_ROWS = 128
NGATH = CI // GATHER_ROWS

_OFFS = ((np.arange(PER * L, dtype=np.int64) % F) * V).astype(np.int32)

_mesh = plsc.VectorSubcoreMesh(
    core_axis_name="c", subcore_axis_name="s", num_cores=NC, num_subcores=NS
)


@functools.partial(
    pl.kernel,
    out_type=jax.ShapeDtypeStruct((R * D,), jnp.float32),
    mesh=_mesh,
    compiler_params=pltpu.CompilerParams(
        needs_layout_passes=False, use_tc_tiling_on_sc=True
    ),
    scratch_types=[
        pltpu.VMEM((D, PIECE), jnp.float32),
        pltpu.VMEM((D, PIECE), jnp.float32),
        pltpu.VMEM((D, PIECE), jnp.float32),
        pltpu.VMEM((PIECE * D,), jnp.float32),
        pltpu.VMEM((PIECE * D,), jnp.float32),
        pltpu.SemaphoreType.DMA,
        pltpu.SemaphoreType.DMA,
        pltpu.SemaphoreType.DMA,
        pltpu.SemaphoreType.DMA,
        pltpu.SemaphoreType.DMA,
    ],
)
def _detile(tt_hbm, last_hbm, lin_hbm, inb0, inb1, inb2, outb0, outb1,
            sin0, sin1, sin2, sout0, sout1):
    wid = lax.axis_index("s") * NC + lax.axis_index("c")
    inbufs, outbufs = (inb0, inb1, inb2), (outb0, outb1)
    sins, souts = (sin0, sin1, sin2), (sout0, sout1)
    base16 = lax.iota(jnp.int32, L) * D
    colidx = [base16 + d for d in range(D)]

    def start_in(k):
        p = wid + k * NW
        return pltpu.async_copy(
            tt_hbm.at[:, pl.ds(p * PIECE, PIECE)], inbufs[k % NIB], sins[k % NIB]
        )

    def transpose(k, nrows=PIECE):
        ib, ob = inbufs[k % NIB], outbufs[k % 2]

        @pl.loop(0, nrows // L)
        def _blk(g):
            rbase = g * L
            dst = ob.at[pl.ds(rbase * D, L * D)]
            vs = [ib[d, pl.ds(rbase, L)] for d in range(D)]
            for d in range(D):
                plsc.store_scatter(dst, [colidx[d]], vs[d])

    def start_out(k):
        p = wid + k * NW
        return pltpu.async_copy(
            outbufs[k % 2], lin_hbm.at[pl.ds(p * PIECE * D, PIECE * D)],
            souts[k % 2],
        )

    h_in = {0: start_in(0)}
    if KFULL > 1:
        h_in[1] = start_in(1)
    h_out = {}
    for k in range(KFULL):
        if k + 2 < KFULL:
            h_in[k + 2] = start_in(k + 2)
        h_in.pop(k).wait()
        if k - 2 in h_out:
            h_out.pop(k - 2).wait()
        transpose(k)
        h_out[k] = start_out(k)

    h_out.pop(KFULL - 2).wait()
    h_out.pop(KFULL - 1).wait()

    @pl.when(wid < KREM)
    def _():
        p = NPIECE - KREM + wid
        pltpu.async_copy(
            tt_hbm.at[:, pl.ds(p * PIECE, PIECE)], inb0, sin0
        ).wait()
        transpose(0)
        pltpu.async_copy(
            outb0, lin_hbm.at[pl.ds(p * PIECE * D, PIECE * D)], sout0
        ).wait()

    @pl.when(wid == KREM)
    def _():
        pltpu.async_copy(
            tt_hbm.at[:, pl.ds(TAILOFF, TAIL)],
            inb0.at[:, pl.ds(0, TAIL)], sin0,
        ).wait()
        transpose(0, nrows=TAIL)
        pltpu.async_copy(
            outb0.at[pl.ds(0, TAIL * D)],
            lin_hbm.at[pl.ds(TAILOFF * D, TAIL * D)], sout0,
        ).wait()

    @pl.when(wid == KREM + 1)
    def _():
        pltpu.async_copy(
            last_hbm, outb0.at[pl.ds(0, LASTN * D)], sin0
        ).wait()
        pltpu.async_copy(
            outb0.at[pl.ds(0, LASTN * D)],
            lin_hbm.at[pl.ds(LASTOFF * D, LASTN * D)], sout0,
        ).wait()


@functools.partial(
    pl.kernel,
    out_type=jax.ShapeDtypeStruct((B,), jnp.float32),
    mesh=_mesh,
    compiler_params=pltpu.CompilerParams(
        needs_layout_passes=False, use_tc_tiling_on_sc=False
    ),
    scratch_types=[
        pltpu.VMEM((PER * L,), jnp.int32),
        pltpu.VMEM((CI,), jnp.int32),
        pltpu.VMEM((CI,), jnp.int32),
        pltpu.VMEM((CI,), jnp.int32),
        pltpu.VMEM((CI, D), jnp.float32),
        pltpu.VMEM((CI, D), jnp.float32),
        pltpu.VMEM((RPW,), jnp.float32),
        pltpu.SemaphoreType.DMA,
        pltpu.SemaphoreType.DMA,
    ],
)
def _fm_cross(table_hbm, x_hbm, offs_hbm, out_hbm,
              offs_v, xbuf, idxb0, idxb1, rows0, rows1, out_v, sem0, sem1):
    wid = lax.axis_index("s") * NC + lax.axis_index("c")
    tbase = wid * (RPW * F)
    pltpu.sync_copy(offs_hbm, offs_v)
    offs = [offs_v[pl.ds(p * L, L)] for p in range(PER)]
    idxbs, rowss, sems = (idxb0, idxb1), (rows0, rows1), (sem0, sem1)

    def build_and_fire(c):
        idxb, rows, sem = idxbs[c % 2], rowss[c % 2], sems[c % 2]
        base = tbase + c * CI
        pltpu.sync_copy(x_hbm.at[pl.ds(base, CI)], xbuf)

        @pl.loop(0, NPERIOD)
        def _b(it):
            xb = it * (PER * L)
            xs = [xbuf[pl.ds(xb + p * L, L)] for p in range(PER)]
            for p in range(PER):
                idxb[pl.ds(xb + p * L, L)] = xs[p] + offs[p]

        for j in range(NGATH):
            pltpu.async_copy(
                table_hbm.at[idxb.at[pl.ds(j * GATHER_ROWS, GATHER_ROWS)]],
                rows.at[pl.ds(j * GATHER_ROWS, GATHER_ROWS)],
                sem,
            )

    def drain(c):
        idxb, rows, sem = idxbs[c % 2], rowss[c % 2], sems[c % 2]
        for j in range(NGATH):
            pltpu.make_async_copy(
                table_hbm.at[idxb.at[pl.ds(j * GATHER_ROWS, GATHER_ROWS)]],
                rows.at[pl.ds(j * GATHER_ROWS, GATHER_ROWS)],
                sem,
            ).wait()

    def compute(c):
        rows = rowss[c % 2]

        @pl.loop(0, CHUNK // L)
        def _rowblk(b):
            lane = lax.iota(jnp.int32, L)
            res = jnp.zeros((L,), jnp.float32)
            for k in range(L):
                rb = (b * L + k) * F
                vs = [rows[rb + f] for f in range(F)]
                s = vs[0]
                ss = vs[0] * vs[0]
                for f in range(1, F):
                    s = s + vs[f]
                    ss = ss + vs[f] * vs[f]
                val = 0.5 * jnp.sum(s * s - ss)
                res = jnp.where(lane == k, val, res)
            out_v[pl.ds(c * CHUNK + b * L, L)] = res

    build_and_fire(0)
    for c in range(NCHUNK):
        if c + 1 < NCHUNK:
            build_and_fire(c + 1)
        drain(c)
        compute(c)

    pltpu.sync_copy(out_v, out_hbm.at[pl.ds(wid * RPW, RPW)])


def kernel(x, table):
    xflat = x.astype(jnp.int32).reshape(B * F)
    last = table[LASTOFF:].reshape(LASTN * D)
    lin = _detile(jnp.swapaxes(table, 0, 1), last)
    out = _fm_cross(lin.reshape(R, D), xflat, jnp.asarray(_OFFS))
    return out.reshape(B, 1)
